# Initial kernel scaffold; baseline (speedup 1.0000x reference)
import jax
import jax.numpy as jnp
from jax.experimental import pallas as pl


def kernel(x, pos, W_qkv, b_qkv, W_out, b_out, W_pos, b_pos, sigma_att, kW1, kb1, kW2, kb2, vW1, vb1, vW2, vb2, qW1, qb1, qW2, qb2, gW, gb):
    raise NotImplementedError("write your pallas kernel here")


# speedup vs baseline: 1.2883x; 1.2883x over previous
import functools
import math

import jax
import jax.numpy as jnp
from jax import lax
from jax.experimental import pallas as pl
from jax.experimental.pallas import tpu as pltpu
from jax.experimental.pallas import tpu_sc as plsc

N = 4096
DIM = 1024
H = 16
DH = 64
BALL = 128
CBS = 8
SEL = 4
PD = 3
NB = N // CBS
M = N // BALL
BPB = BALL // CBS
HID = 2 * CBS * DH
CD = CBS * DH
PPAD = 128
GPAD = 128

f32 = jnp.float32
_PREC = lax.Precision.HIGHEST

_R1 = 512
_CT1 = 768


def _prologue_body(pos_ref, x_ref, wpos_ref, bpos_ref, wqkv_ref, bqkv_ref,
                   gw_ref, gb_ref, qkv_ref, gates_ref):
    j = pl.program_id(1)
    pr = pos_ref[...]
    ri = lax.broadcasted_iota(jnp.int32, (_R1, _R1), 0) // BALL
    ci = lax.broadcasted_iota(jnp.int32, (_R1, _R1), 1) // BALL
    avg = jnp.where(ri == ci, 1.0 / BALL, 0.0).astype(f32)
    rel = pr - jnp.dot(avg, pr, preferred_element_type=f32, precision=_PREC)
    xp = x_ref[...] + jnp.dot(rel, wpos_ref[...], preferred_element_type=f32) \
        + bpos_ref[...]
    qkv_ref[...] = jnp.dot(xp, wqkv_ref[...], preferred_element_type=f32) \
        + bqkv_ref[...]

    @pl.when(j == 0)
    def _():
        gates_ref[...] = jax.nn.sigmoid(
            jnp.dot(xp, gw_ref[...], preferred_element_type=f32) + gb_ref[...])


def _local_body(pos_ref, qkv_ref, sig_ref, out_ref):
    pf = pos_ref[...]
    gram = lax.dot_general(pf, pf, (((1,), (1,)), ((), ())),
                           preferred_element_type=f32, precision=_PREC)
    ri = lax.broadcasted_iota(jnp.int32, (BALL, BALL), 0)
    ci = lax.broadcasted_iota(jnp.int32, (BALL, BALL), 1)
    gd = jnp.where(ri == ci, gram, 0.0)
    diag_c = jnp.sum(gd, axis=1, keepdims=True)
    diag_r = jnp.sum(gd, axis=0, keepdims=True)
    dist = jnp.sqrt(jnp.maximum(diag_c + diag_r - 2.0 * gram, 0.0))
    for h in range(H):
        q = qkv_ref[:, h * DH:(h + 1) * DH]
        k = qkv_ref[:, DIM + h * DH:DIM + (h + 1) * DH]
        v = qkv_ref[:, 2 * DIM + h * DH:2 * DIM + (h + 1) * DH]
        s = lax.dot_general(q, k, (((1,), (1,)), ((), ())),
                            preferred_element_type=f32) * 0.125
        s = s + sig_ref[0, h] * dist
        s = s - jnp.max(s, axis=1, keepdims=True)
        e = jnp.exp(s)
        p = e / jnp.sum(e, axis=1, keepdims=True)
        out_ref[:, h * DH:(h + 1) * DH] = jnp.dot(p, v,
                                                  preferred_element_type=f32)


def _cmlp_body(mat_ref, w1_ref, b1_ref, w2_ref, b2_ref, out_ref):
    hmat = jnp.maximum(
        jnp.dot(mat_ref[0], w1_ref[0], preferred_element_type=f32)
        + b1_ref[0], 0.0)
    out_ref[0] = jnp.dot(hmat, w2_ref[0], preferred_element_type=f32) \
        + b2_ref[0]


def _coarse_body(cq_ref, ck_ref, cv_ref, co_ref, idx_ref):
    h = pl.program_id(0)
    cq = cq_ref[0]
    ck = ck_ref[0]
    cv = cv_ref[0]
    s = lax.dot_general(cq, ck, (((1,), (1,)), ((), ())),
                        preferred_element_type=f32) * 0.125
    sm0 = s - jnp.max(s, axis=1, keepdims=True)
    e = jnp.exp(sm0)
    co_ref[0] = jnp.dot(e / jnp.sum(e, axis=1, keepdims=True), cv,
                        preferred_element_type=f32)
    ri = lax.broadcasted_iota(jnp.int32, (NB, NB), 0) // BPB
    ci_b = lax.broadcasted_iota(jnp.int32, (NB, NB), 1) // BPB
    neg = jnp.float32(-jnp.inf)
    sm = jnp.where(ri == ci_b, neg, s)
    cidx = lax.broadcasted_iota(jnp.int32, (NB, NB), 1)
    cols = []
    for _ in range(SEL):
        m = jnp.max(sm, axis=1, keepdims=True)
        idxv = jnp.min(jnp.where(sm == m, cidx, NB), axis=1, keepdims=True)
        cols.append(idxv + h * NB)
        sm = jnp.where(cidx == idxv, neg, sm)
    idx_ref[0] = jnp.concatenate(cols, axis=1)


def _fine_body(q_ref, sk_ref, sv_ref, out_ref):
    q = q_ref[0]
    k = sk_ref[0]
    v = sv_ref[0]
    s = lax.dot_general(q, k, (((1,), (1,)), ((), ())),
                        preferred_element_type=f32) * 0.125
    ri = lax.broadcasted_iota(jnp.int32, (BALL, SEL * CBS * BPB), 0) // CBS
    ci = lax.broadcasted_iota(jnp.int32, (BALL, SEL * CBS * BPB), 1) \
        // (SEL * CBS)
    s = jnp.where(ri == ci, s, -jnp.inf)
    s = s - jnp.max(s, axis=1, keepdims=True)
    e = jnp.exp(s)
    out_ref[0] = jnp.dot(e / jnp.sum(e, axis=1, keepdims=True), v,
                         preferred_element_type=f32)


def _fuse_body(lo_ref, co_ref, fi_ref, g_ref, e8_ref, el_ref, ec_ref, ef_ref,
               wout_ref, bout_ref, out_ref):
    g = g_ref[...]
    lo = lo_ref[...]
    fi = fi_ref[...]
    cos = [jnp.dot(e8_ref[...], co_ref[h], preferred_element_type=f32, precision=_PREC)
           for h in range(H)]
    co = jnp.concatenate(cos, axis=1)
    gl = jnp.dot(g, el_ref[...], preferred_element_type=f32, precision=_PREC)
    gc = jnp.dot(g, ec_ref[...], preferred_element_type=f32, precision=_PREC)
    gf = jnp.dot(g, ef_ref[...], preferred_element_type=f32, precision=_PREC)
    fused = gl * lo + gc * co + gf * fi
    out_ref[...] = jnp.dot(fused, wout_ref[...], preferred_element_type=f32) \
        + bout_ref[...]


def _sc_gather(kmat, vmat, idx_flat):
    info = plsc.get_sparse_core_info()
    nw = info.num_cores * info.num_subcores
    total = idx_flat.shape[0]
    per_w = total // nw
    ch = 64
    nch = per_w // ch
    mesh = plsc.VectorSubcoreMesh(core_axis_name="c", subcore_axis_name="s")

    @functools.partial(
        pl.kernel, mesh=mesh,
        out_type=[jax.ShapeDtypeStruct((total, CD), f32),
                  jax.ShapeDtypeStruct((total, CD), f32)],
        scratch_types=[pltpu.VMEM((ch,), jnp.int32),
                       pltpu.VMEM((ch, CD), f32),
                       pltpu.VMEM((ch, CD), f32),
                       pltpu.SemaphoreType.DMA,
                       pltpu.SemaphoreType.DMA],
    )
    def gather(k_hbm, v_hbm, idx_hbm, selk_hbm, selv_hbm,
               idx_v, krows, vrows, ksem, vsem):
        wid = lax.axis_index("s") * info.num_cores + lax.axis_index("c")
        base = wid * per_w
        for c in range(nch):
            off = base + c * ch
            pltpu.sync_copy(idx_hbm.at[pl.ds(off, ch)], idx_v)
            cpk = pltpu.async_copy(k_hbm.at[idx_v], krows, ksem)
            cpv = pltpu.async_copy(v_hbm.at[idx_v], vrows, vsem)
            cpk.wait()
            pltpu.sync_copy(krows, selk_hbm.at[pl.ds(off, ch)])
            cpv.wait()
            pltpu.sync_copy(vrows, selv_hbm.at[pl.ds(off, ch)])

    return gather(kmat, vmat, idx_flat)


def kernel(x, pos, W_qkv, b_qkv, W_out, b_out, W_pos, b_pos, sigma_att,
           kW1, kb1, kW2, kb2, vW1, vb1, vW2, vb2, qW1, qb1, qW2, qb2, gW, gb):
    x2 = x[0]
    pos_p = jnp.pad(pos[0], ((0, 0), (0, PPAD - PD)))
    wpos_p = jnp.pad(W_pos, ((0, PPAD - PD), (0, 0)))
    gw_p = jnp.pad(gW, ((0, 0), (0, GPAD - 3 * H)))
    gb_p = jnp.pad(gb, (0, GPAD - 3 * H)).reshape(1, GPAD)
    sig = sigma_att.reshape(1, H)

    qkv, gates = pl.pallas_call(
        _prologue_body,
        grid=(N // _R1, 3 * DIM // _CT1),
        in_specs=[
            pl.BlockSpec((_R1, PPAD), lambda i, j: (i, 0)),
            pl.BlockSpec((_R1, DIM), lambda i, j: (i, 0)),
            pl.BlockSpec((PPAD, DIM), lambda i, j: (0, 0)),
            pl.BlockSpec((1, DIM), lambda i, j: (0, 0)),
            pl.BlockSpec((DIM, _CT1), lambda i, j: (0, j)),
            pl.BlockSpec((1, _CT1), lambda i, j: (0, j)),
            pl.BlockSpec((DIM, GPAD), lambda i, j: (0, 0)),
            pl.BlockSpec((1, GPAD), lambda i, j: (0, 0)),
        ],
        out_specs=[
            pl.BlockSpec((_R1, _CT1), lambda i, j: (i, j)),
            pl.BlockSpec((_R1, GPAD), lambda i, j: (i, 0)),
        ],
        out_shape=[
            jax.ShapeDtypeStruct((N, 3 * DIM), f32),
            jax.ShapeDtypeStruct((N, GPAD), f32),
        ],
    )(pos_p, x2, wpos_p, b_pos.reshape(1, DIM), W_qkv,
      b_qkv.reshape(1, 3 * DIM), gw_p, gb_p)

    local = pl.pallas_call(
        _local_body,
        grid=(M,),
        in_specs=[
            pl.BlockSpec((BALL, PPAD), lambda b: (b, 0)),
            pl.BlockSpec((BALL, 3 * DIM), lambda b: (b, 0)),
            pl.BlockSpec((1, H), lambda b: (0, 0)),
        ],
        out_specs=pl.BlockSpec((BALL, DIM), lambda b: (b, 0)),
        out_shape=jax.ShapeDtypeStruct((N, DIM), f32),
    )(pos_p, qkv, sig)

    kh = qkv[:, DIM:2 * DIM].reshape(N, H, DH).transpose(1, 0, 2)
    vh = qkv[:, 2 * DIM:].reshape(N, H, DH).transpose(1, 0, 2)
    qh = qkv[:, :DIM].reshape(N, H, DH).transpose(1, 0, 2)
    kmat = kh.reshape(H * NB, CD)
    vmat = vh.reshape(H * NB, CD)
    qmat = qh.reshape(H * NB, CD)
    mats = jnp.stack([kmat, vmat, qmat])
    w1s = jnp.stack([kW1, vW1, qW1])
    b1s = jnp.stack([kb1, vb1, qb1]).reshape(3, 1, HID)
    w2s = jnp.stack([kW2, vW2, qW2])
    b2s = jnp.stack([kb2, vb2, qb2]).reshape(3, 1, DH)

    couts = pl.pallas_call(
        _cmlp_body,
        grid=(3, H),
        in_specs=[
            pl.BlockSpec((1, NB, CD), lambda t, i: (t, i, 0)),
            pl.BlockSpec((1, CD, HID), lambda t, i: (t, 0, 0)),
            pl.BlockSpec((1, 1, HID), lambda t, i: (t, 0, 0)),
            pl.BlockSpec((1, HID, DH), lambda t, i: (t, 0, 0)),
            pl.BlockSpec((1, 1, DH), lambda t, i: (t, 0, 0)),
        ],
        out_specs=pl.BlockSpec((1, NB, DH), lambda t, i: (t, i, 0)),
        out_shape=jax.ShapeDtypeStruct((3, H * NB, DH), f32),
    )(mats, w1s, b1s, w2s, b2s)
    ck3 = couts[0].reshape(H, NB, DH)
    cv3 = couts[1].reshape(H, NB, DH)
    cq3 = couts[2].reshape(H, NB, DH)

    co_b, idx3 = pl.pallas_call(
        _coarse_body,
        grid=(H,),
        in_specs=[
            pl.BlockSpec((1, NB, DH), lambda h: (h, 0, 0)),
            pl.BlockSpec((1, NB, DH), lambda h: (h, 0, 0)),
            pl.BlockSpec((1, NB, DH), lambda h: (h, 0, 0)),
        ],
        out_specs=[
            pl.BlockSpec((1, NB, DH), lambda h: (h, 0, 0)),
            pl.BlockSpec((1, NB, SEL), lambda h: (h, 0, 0)),
        ],
        out_shape=[
            jax.ShapeDtypeStruct((H, NB, DH), f32),
            jax.ShapeDtypeStruct((H, NB, SEL), jnp.int32),
        ],
    )(cq3, ck3, cv3)

    idx_flat = idx3.reshape(H * NB * SEL)
    selk, selv = _sc_gather(kmat, vmat, idx_flat)
    skr = selk.reshape(H, NB * SEL * CBS, DH)
    svr = selv.reshape(H, NB * SEL * CBS, DH)

    fine_hm = pl.pallas_call(
        _fine_body,
        grid=(H, M),
        in_specs=[
            pl.BlockSpec((1, BALL, DH), lambda h, g: (h, g, 0)),
            pl.BlockSpec((1, SEL * CBS * BPB, DH), lambda h, g: (h, g, 0)),
            pl.BlockSpec((1, SEL * CBS * BPB, DH), lambda h, g: (h, g, 0)),
        ],
        out_specs=pl.BlockSpec((1, BALL, DH), lambda h, g: (h, g, 0)),
        out_shape=jax.ShapeDtypeStruct((H, N, DH), f32),
    )(qh, skr, svr)
    fine = fine_hm.transpose(1, 0, 2).reshape(N, DIM)

    e8 = (jnp.arange(_R1)[:, None] // CBS
          == jnp.arange(_R1 // CBS)[None, :]).astype(f32)
    hcol = jnp.arange(DIM) // DH
    sels = [(jnp.arange(GPAD)[:, None] == 3 * hcol[None, :] + j).astype(f32)
            for j in range(3)]

    out2 = pl.pallas_call(
        _fuse_body,
        grid=(N // _R1,),
        in_specs=[
            pl.BlockSpec((_R1, DIM), lambda i: (i, 0)),
            pl.BlockSpec((H, _R1 // CBS, DH), lambda i: (0, i, 0)),
            pl.BlockSpec((_R1, DIM), lambda i: (i, 0)),
            pl.BlockSpec((_R1, GPAD), lambda i: (i, 0)),
            pl.BlockSpec((_R1, _R1 // CBS), lambda i: (0, 0)),
            pl.BlockSpec((GPAD, DIM), lambda i: (0, 0)),
            pl.BlockSpec((GPAD, DIM), lambda i: (0, 0)),
            pl.BlockSpec((GPAD, DIM), lambda i: (0, 0)),
            pl.BlockSpec((DIM, DIM), lambda i: (0, 0)),
            pl.BlockSpec((1, DIM), lambda i: (0, 0)),
        ],
        out_specs=pl.BlockSpec((_R1, DIM), lambda i: (i, 0)),
        out_shape=jax.ShapeDtypeStruct((N, DIM), f32),
    )(local, co_b, fine, gates, e8, sels[0], sels[1], sels[2],
      W_out, b_out.reshape(1, DIM))

    return out2.reshape(1, N, DIM)
